# baseline (speedup 1.0000x reference)
import functools

import jax
import jax.numpy as jnp
from jax import lax
from jax.experimental import pallas as pl
from jax.experimental.pallas import tpu as pltpu
from jax.experimental.pallas import tpu_sc as plsc

T = 2048
H = 768
E = 8
TOPK = 2
DFF = 384
NG = 4
TG = 2
RSF = 2.5

BT = 256

_NC = 2
_NS = 16
_TPW = 128
_NWU = T // _TPW
_L = 16


def _router_head_body(gw_ref, x_ref, bias_ref, scores_ref, s_ref):
    logits_t = jax.lax.dot_general(
        gw_ref[...], x_ref[...], (((1,), (1,)), ((), ())),
        preferred_element_type=jnp.float32)
    scores = jax.nn.sigmoid(logits_t)
    scores_ref[...] = scores
    s_ref[...] = scores + bias_ref[...]


def _sc_topk_body(scores_hbm, s_hbm, w_hbm, scores_v, s_v, w_v):
    wid = lax.axis_index("s") * _NC + lax.axis_index("c")

    @pl.when(wid < _NWU)
    def _active():
        _sc_topk_worker(wid, scores_hbm, s_hbm, w_hbm, scores_v, s_v, w_v)


def _sc_topk_worker(wid, scores_hbm, s_hbm, w_hbm, scores_v, s_v, w_v):
    base = wid * _TPW
    pltpu.sync_copy(scores_hbm.at[:, pl.ds(base, _TPW)], scores_v)
    pltpu.sync_copy(s_hbm.at[:, pl.ds(base, _TPW)], s_v)
    for cpos in range(_TPW // _L):
        sl = pl.ds(cpos * _L, _L)
        s = [s_v[e, sl] for e in range(E)]
        sc = [scores_v[e, sl] for e in range(E)]
        g = [s[2 * j] + s[2 * j + 1] for j in range(NG)]
        one = jnp.full((_L,), 1.0, jnp.float32)
        zero = jnp.zeros((_L,), jnp.float32)
        gsel = []
        for j in range(NG):
            rank = zero
            for kk in range(NG):
                if kk == j:
                    continue
                beats = g[kk] > g[j]
                if kk < j:
                    beats = beats | (g[kk] == g[j])
                rank = rank + jnp.where(beats, one, zero)
            gsel.append(rank < TG)
        tmp = [jnp.where(gsel[e // (E // NG)], s[e], zero) for e in range(E)]
        esel = []
        for j in range(E):
            rank = zero
            for kk in range(E):
                if kk == j:
                    continue
                beats = tmp[kk] > tmp[j]
                if kk < j:
                    beats = beats | (tmp[kk] == tmp[j])
                rank = rank + jnp.where(beats, one, zero)
            esel.append(rank < TOPK)
        w = [jnp.where(esel[e], sc[e], zero) for e in range(E)]
        denom = w[0]
        for e in range(1, E):
            denom = denom + w[e]
        for e in range(E):
            w_v[e, sl] = w[e] / denom
    pltpu.sync_copy(w_v, w_hbm.at[:, pl.ds(base, _TPW)])


_sc_topk = functools.partial(
    pl.kernel,
    out_type=jax.ShapeDtypeStruct((E, T), jnp.float32),
    scratch_types=[
        pltpu.VMEM((E, _TPW), jnp.float32),
        pltpu.VMEM((E, _TPW), jnp.float32),
        pltpu.VMEM((E, _TPW), jnp.float32),
    ],
    mesh=plsc.VectorSubcoreMesh(core_axis_name="c", subcore_axis_name="s"),
)(_sc_topk_body)


def _moe_body(x_ref, wt_ref, wgu_ref, sgu_ref, wd_ref, sd_ref, out_ref):
    x = x_ref[...]
    gu_all = jax.lax.dot_general(
        x, wgu_ref[...], (((1,), (1,)), ((), ())),
        preferred_element_type=jnp.float32)
    sgu = jax.lax.dot_general(
        x, sgu_ref[...], (((1,), (1,)), ((), ())),
        preferred_element_type=jnp.float32)

    w_full = wt_ref[...].T * RSF

    sgate = sgu[:, :DFF]
    sup = sgu[:, DFF:]
    sh = (sgate * jax.nn.sigmoid(sgate) * sup).astype(jnp.bfloat16)
    acc = jax.lax.dot_general(
        sh, sd_ref[...], (((1,), (1,)), ((), ())),
        preferred_element_type=jnp.float32)
    for e in range(E):
        gate = gu_all[:, e * 2 * DFF:e * 2 * DFF + DFF]
        up = gu_all[:, e * 2 * DFF + DFF:(e + 1) * 2 * DFF]
        h = (gate * jax.nn.sigmoid(gate) * up
             * w_full[:, e:e + 1]).astype(jnp.bfloat16)
        acc = acc + jax.lax.dot_general(
            h, wd_ref[e], (((1,), (1,)), ((), ())),
            preferred_element_type=jnp.float32)
    out_ref[...] = acc


@jax.jit
def kernel(hidden_states, gate_W, e_score_correction_bias, We_gate_up,
           We_down, Ws_gate_up, Ws_down):
    bias_col = e_score_correction_bias.reshape(E, 1)
    wgu2d = We_gate_up.reshape(E * 2 * DFF, H)

    scores_t, s_t = pl.pallas_call(
        _router_head_body,
        out_shape=(jax.ShapeDtypeStruct((E, T), jnp.float32),
                   jax.ShapeDtypeStruct((E, T), jnp.float32)),
    )(gate_W, hidden_states, bias_col)

    w_t = _sc_topk(scores_t, s_t)

    grid = (T // BT,)
    return pl.pallas_call(
        _moe_body,
        grid=grid,
        in_specs=[
            pl.BlockSpec((BT, H), lambda i: (i, 0)),
            pl.BlockSpec((E, BT), lambda i: (0, i)),
            pl.BlockSpec((E * 2 * DFF, H), lambda i: (0, 0)),
            pl.BlockSpec((2 * DFF, H), lambda i: (0, 0)),
            pl.BlockSpec((E, H, DFF), lambda i: (0, 0, 0)),
            pl.BlockSpec((H, DFF), lambda i: (0, 0)),
        ],
        out_specs=pl.BlockSpec((BT, H), lambda i: (i, 0)),
        out_shape=jax.ShapeDtypeStruct((T, H), jnp.float32),
        compiler_params=pltpu.CompilerParams(
            dimension_semantics=("arbitrary",),
        ),
    )(hidden_states, w_t, wgu2d, Ws_gate_up, We_down, Ws_down)


# speedup vs baseline: 1.0675x; 1.0675x over previous
import functools

import jax
import jax.numpy as jnp
from jax import lax
from jax.experimental import pallas as pl
from jax.experimental.pallas import tpu as pltpu
from jax.experimental.pallas import tpu_sc as plsc

T = 2048
H = 768
E = 8
TOPK = 2
DFF = 384
NG = 4
TG = 2
RSF = 2.5

BT = 256

_NC = 2
_NS = 16
_TPW = 128
_NWU = T // _TPW
_L = 16


def _router_head_body(gw_ref, x_ref, bias_ref, scores_ref, s_ref):
    logits_t = jax.lax.dot_general(
        gw_ref[...], x_ref[...], (((1,), (1,)), ((), ())),
        preferred_element_type=jnp.float32)
    scores = jax.nn.sigmoid(logits_t)
    scores_ref[...] = scores
    s_ref[...] = scores + bias_ref[...]


def _sc_topk_body(scores_hbm, s_hbm, w_hbm, scores_v, s_v, w_v):
    wid = lax.axis_index("s") * _NC + lax.axis_index("c")

    @pl.when(wid < _NWU)
    def _active():
        _sc_topk_worker(wid, scores_hbm, s_hbm, w_hbm, scores_v, s_v, w_v)


def _sc_topk_worker(wid, scores_hbm, s_hbm, w_hbm, scores_v, s_v, w_v):
    base = wid * _TPW
    pltpu.sync_copy(scores_hbm.at[:, pl.ds(base, _TPW)], scores_v)
    pltpu.sync_copy(s_hbm.at[:, pl.ds(base, _TPW)], s_v)
    for cpos in range(_TPW // _L):
        sl = pl.ds(cpos * _L, _L)
        s = [s_v[e, sl] for e in range(E)]
        sc = [scores_v[e, sl] for e in range(E)]
        g = [s[2 * j] + s[2 * j + 1] for j in range(NG)]
        one = jnp.full((_L,), 1.0, jnp.float32)
        zero = jnp.zeros((_L,), jnp.float32)
        gsel = []
        for j in range(NG):
            rank = zero
            for kk in range(NG):
                if kk == j:
                    continue
                beats = g[kk] > g[j]
                if kk < j:
                    beats = beats | (g[kk] == g[j])
                rank = rank + jnp.where(beats, one, zero)
            gsel.append(rank < TG)
        tmp = [jnp.where(gsel[e // (E // NG)], s[e], zero) for e in range(E)]
        esel = []
        for j in range(E):
            rank = zero
            for kk in range(E):
                if kk == j:
                    continue
                beats = tmp[kk] > tmp[j]
                if kk < j:
                    beats = beats | (tmp[kk] == tmp[j])
                rank = rank + jnp.where(beats, one, zero)
            esel.append(rank < TOPK)
        w = [jnp.where(esel[e], sc[e], zero) for e in range(E)]
        denom = w[0]
        for e in range(1, E):
            denom = denom + w[e]
        for e in range(E):
            w_v[e, sl] = w[e] / denom
    pltpu.sync_copy(w_v, w_hbm.at[:, pl.ds(base, _TPW)])


_sc_topk = functools.partial(
    pl.kernel,
    out_type=jax.ShapeDtypeStruct((E, T), jnp.float32),
    scratch_types=[
        pltpu.VMEM((E, _TPW), jnp.float32),
        pltpu.VMEM((E, _TPW), jnp.float32),
        pltpu.VMEM((E, _TPW), jnp.float32),
    ],
    mesh=plsc.VectorSubcoreMesh(core_axis_name="c", subcore_axis_name="s"),
)(_sc_topk_body)


def _shared_body(x_ref, sgu_ref, sd_ref, out_ref):
    x = x_ref[...]
    sgu = jax.lax.dot_general(
        x, sgu_ref[...], (((1,), (1,)), ((), ())),
        preferred_element_type=jnp.float32)
    sgate = sgu[:, :DFF]
    sup = sgu[:, DFF:]
    sh = (sgate * jax.nn.sigmoid(sgate) * sup).astype(jnp.bfloat16)
    out_ref[...] = jax.lax.dot_general(
        sh, sd_ref[...], (((1,), (1,)), ((), ())),
        preferred_element_type=jnp.float32)


def _moe_body(x_ref, wt_ref, shared_ref, wgu_ref, wd_ref, out_ref):
    x = x_ref[...]
    gu_all = jax.lax.dot_general(
        x, wgu_ref[...], (((1,), (1,)), ((), ())),
        preferred_element_type=jnp.float32)

    w_full = wt_ref[...].T * RSF

    acc = shared_ref[...]
    for e in range(E):
        gate = gu_all[:, e * 2 * DFF:e * 2 * DFF + DFF]
        up = gu_all[:, e * 2 * DFF + DFF:(e + 1) * 2 * DFF]
        h = (gate * jax.nn.sigmoid(gate) * up
             * w_full[:, e:e + 1]).astype(jnp.bfloat16)
        acc = acc + jax.lax.dot_general(
            h, wd_ref[e], (((1,), (1,)), ((), ())),
            preferred_element_type=jnp.float32)
    out_ref[...] = acc


@jax.jit
def kernel(hidden_states, gate_W, e_score_correction_bias, We_gate_up,
           We_down, Ws_gate_up, Ws_down):
    bias_col = e_score_correction_bias.reshape(E, 1)
    wgu2d = We_gate_up.reshape(E * 2 * DFF, H)

    scores_t, s_t = pl.pallas_call(
        _router_head_body,
        out_shape=(jax.ShapeDtypeStruct((E, T), jnp.float32),
                   jax.ShapeDtypeStruct((E, T), jnp.float32)),
    )(gate_W, hidden_states, bias_col)

    w_t = _sc_topk(scores_t, s_t)

    shared_out = pl.pallas_call(
        _shared_body,
        grid=(T // 512,),
        in_specs=[
            pl.BlockSpec((512, H), lambda i: (i, 0)),
            pl.BlockSpec((2 * DFF, H), lambda i: (0, 0)),
            pl.BlockSpec((H, DFF), lambda i: (0, 0)),
        ],
        out_specs=pl.BlockSpec((512, H), lambda i: (i, 0)),
        out_shape=jax.ShapeDtypeStruct((T, H), jnp.float32),
        compiler_params=pltpu.CompilerParams(
            dimension_semantics=("arbitrary",),
        ),
    )(hidden_states, Ws_gate_up, Ws_down)

    grid = (T // BT,)
    return pl.pallas_call(
        _moe_body,
        grid=grid,
        in_specs=[
            pl.BlockSpec((BT, H), lambda i: (i, 0)),
            pl.BlockSpec((E, BT), lambda i: (0, i)),
            pl.BlockSpec((BT, H), lambda i: (i, 0)),
            pl.BlockSpec((E * 2 * DFF, H), lambda i: (0, 0)),
            pl.BlockSpec((E, H, DFF), lambda i: (0, 0, 0)),
        ],
        out_specs=pl.BlockSpec((BT, H), lambda i: (i, 0)),
        out_shape=jax.ShapeDtypeStruct((T, H), jnp.float32),
        compiler_params=pltpu.CompilerParams(
            dimension_semantics=("arbitrary",),
        ),
    )(hidden_states, w_t, shared_out, wgu2d, We_down)


# speedup vs baseline: 1.0687x; 1.0012x over previous
import functools

import jax
import jax.numpy as jnp
from jax import lax
from jax.experimental import pallas as pl
from jax.experimental.pallas import tpu as pltpu
from jax.experimental.pallas import tpu_sc as plsc

T = 2048
H = 768
E = 8
TOPK = 2
DFF = 384
NG = 4
TG = 2
RSF = 2.5

BT = 256

_NC = 2
_NS = 16
_TPW = 128
_NWU = T // _TPW
_L = 16


def _head_shared_body(x_ref, gw_ref, bias_ref, sgu_ref, sd_ref,
                      shared_ref, scores_ref, s_ref):
    x = x_ref[...]
    logits_t = jax.lax.dot_general(
        gw_ref[...], x, (((1,), (1,)), ((), ())),
        preferred_element_type=jnp.float32)
    scores = jax.nn.sigmoid(logits_t)
    scores_ref[...] = scores
    s_ref[...] = scores + bias_ref[...]
    sgu = jax.lax.dot_general(
        x, sgu_ref[...], (((1,), (1,)), ((), ())),
        preferred_element_type=jnp.float32)
    sgate = sgu[:, :DFF]
    sup = sgu[:, DFF:]
    sh = (sgate * jax.nn.sigmoid(sgate) * sup).astype(jnp.bfloat16)
    shared_ref[...] = jax.lax.dot_general(
        sh, sd_ref[...], (((1,), (1,)), ((), ())),
        preferred_element_type=jnp.float32)


def _sc_topk_body(scores_hbm, s_hbm, w_hbm, scores_v, s_v, w_v):
    wid = lax.axis_index("s") * _NC + lax.axis_index("c")

    @pl.when(wid < _NWU)
    def _active():
        _sc_topk_worker(wid, scores_hbm, s_hbm, w_hbm, scores_v, s_v, w_v)


def _sc_topk_worker(wid, scores_hbm, s_hbm, w_hbm, scores_v, s_v, w_v):
    base = wid * _TPW
    pltpu.sync_copy(scores_hbm.at[:, pl.ds(base, _TPW)], scores_v)
    pltpu.sync_copy(s_hbm.at[:, pl.ds(base, _TPW)], s_v)
    for cpos in range(_TPW // _L):
        sl = pl.ds(cpos * _L, _L)
        s = [s_v[e, sl] for e in range(E)]
        sc = [scores_v[e, sl] for e in range(E)]
        g = [s[2 * j] + s[2 * j + 1] for j in range(NG)]
        one = jnp.full((_L,), 1.0, jnp.float32)
        zero = jnp.zeros((_L,), jnp.float32)
        gsel = []
        for j in range(NG):
            rank = zero
            for kk in range(NG):
                if kk == j:
                    continue
                beats = g[kk] > g[j]
                if kk < j:
                    beats = beats | (g[kk] == g[j])
                rank = rank + jnp.where(beats, one, zero)
            gsel.append(rank < TG)
        tmp = [jnp.where(gsel[e // (E // NG)], s[e], zero) for e in range(E)]
        esel = []
        for j in range(E):
            rank = zero
            for kk in range(E):
                if kk == j:
                    continue
                beats = tmp[kk] > tmp[j]
                if kk < j:
                    beats = beats | (tmp[kk] == tmp[j])
                rank = rank + jnp.where(beats, one, zero)
            esel.append(rank < TOPK)
        w = [jnp.where(esel[e], sc[e], zero) for e in range(E)]
        denom = w[0]
        for e in range(1, E):
            denom = denom + w[e]
        for e in range(E):
            w_v[e, sl] = w[e] / denom
    pltpu.sync_copy(w_v, w_hbm.at[:, pl.ds(base, _TPW)])


_sc_topk = functools.partial(
    pl.kernel,
    out_type=jax.ShapeDtypeStruct((E, T), jnp.float32),
    scratch_types=[
        pltpu.VMEM((E, _TPW), jnp.float32),
        pltpu.VMEM((E, _TPW), jnp.float32),
        pltpu.VMEM((E, _TPW), jnp.float32),
    ],
    mesh=plsc.VectorSubcoreMesh(core_axis_name="c", subcore_axis_name="s"),
)(_sc_topk_body)


def _moe_body(x_ref, wt_ref, shared_ref, wgu_ref, wd_ref, out_ref):
    x = x_ref[...]
    gu_all = jax.lax.dot_general(
        x, wgu_ref[...], (((1,), (1,)), ((), ())),
        preferred_element_type=jnp.float32)

    w_full = wt_ref[...].T * RSF

    acc = shared_ref[...]
    for e in range(E):
        gate = gu_all[:, e * 2 * DFF:e * 2 * DFF + DFF]
        up = gu_all[:, e * 2 * DFF + DFF:(e + 1) * 2 * DFF]
        h = (gate * jax.nn.sigmoid(gate) * up
             * w_full[:, e:e + 1]).astype(jnp.bfloat16)
        acc = acc + jax.lax.dot_general(
            h, wd_ref[e], (((1,), (1,)), ((), ())),
            preferred_element_type=jnp.float32)
    out_ref[...] = acc


@jax.jit
def kernel(hidden_states, gate_W, e_score_correction_bias, We_gate_up,
           We_down, Ws_gate_up, Ws_down):
    bias_col = e_score_correction_bias.reshape(E, 1)
    wgu2d = We_gate_up.reshape(E * 2 * DFF, H)

    BS = 512
    shared_out, scores_t, s_t = pl.pallas_call(
        _head_shared_body,
        grid=(T // BS,),
        in_specs=[
            pl.BlockSpec((BS, H), lambda i: (i, 0)),
            pl.BlockSpec((E, H), lambda i: (0, 0)),
            pl.BlockSpec((E, 1), lambda i: (0, 0)),
            pl.BlockSpec((2 * DFF, H), lambda i: (0, 0)),
            pl.BlockSpec((H, DFF), lambda i: (0, 0)),
        ],
        out_specs=(pl.BlockSpec((BS, H), lambda i: (i, 0)),
                   pl.BlockSpec((E, BS), lambda i: (0, i)),
                   pl.BlockSpec((E, BS), lambda i: (0, i))),
        out_shape=(jax.ShapeDtypeStruct((T, H), jnp.float32),
                   jax.ShapeDtypeStruct((E, T), jnp.float32),
                   jax.ShapeDtypeStruct((E, T), jnp.float32)),
        compiler_params=pltpu.CompilerParams(
            dimension_semantics=("arbitrary",),
        ),
    )(hidden_states, gate_W, bias_col, Ws_gate_up, Ws_down)

    w_t = _sc_topk(scores_t, s_t)

    grid = (T // BT,)
    return pl.pallas_call(
        _moe_body,
        grid=grid,
        in_specs=[
            pl.BlockSpec((BT, H), lambda i: (i, 0)),
            pl.BlockSpec((E, BT), lambda i: (0, i)),
            pl.BlockSpec((BT, H), lambda i: (i, 0)),
            pl.BlockSpec((E * 2 * DFF, H), lambda i: (0, 0)),
            pl.BlockSpec((E, H, DFF), lambda i: (0, 0, 0)),
        ],
        out_specs=pl.BlockSpec((BT, H), lambda i: (i, 0)),
        out_shape=jax.ShapeDtypeStruct((T, H), jnp.float32),
        compiler_params=pltpu.CompilerParams(
            dimension_semantics=("arbitrary",),
        ),
    )(hidden_states, w_t, shared_out, wgu2d, We_down)
